# baseline (device time: 31755 ns/iter reference)
import jax
import jax.numpy as jnp
from jax import lax
from jax.experimental import pallas as pl
from jax.experimental.pallas import tpu as pltpu

N_DEV = 4


def kernel(A, B):
    M, K = A.shape
    _, N = B.shape
    m_per = M // N_DEV

    def body(a_ref, b_ref, out_ref, acc_ref, comm_ref, send_sems, recv_sems):
        my = lax.axis_index("i")
        left = (my - 1) % N_DEV
        right = (my + 1) % N_DEV

        barrier_sem = pltpu.get_barrier_semaphore()
        for nbr in [left, right]:
            pl.semaphore_signal(
                barrier_sem, inc=1,
                device_id=(nbr,), device_id_type=pl.DeviceIdType.MESH,
            )
        pl.semaphore_wait(barrier_sem, 2)

        a_bf = a_ref[...].astype(jnp.bfloat16)
        b_bf = b_ref[...].astype(jnp.bfloat16)
        acc_ref[...] = jnp.dot(a_bf, b_bf, preferred_element_type=jnp.float32)

        first = (my - 1) % N_DEV
        comm_ref[0, :, :] = acc_ref[pl.ds(first * m_per, m_per), :]

        for s in range(N_DEV - 1):
            rdma = pltpu.make_async_remote_copy(
                src_ref=comm_ref.at[s],
                dst_ref=comm_ref.at[s + 1],
                send_sem=send_sems.at[s],
                recv_sem=recv_sems.at[s],
                device_id=(right,),
                device_id_type=pl.DeviceIdType.MESH,
            )
            rdma.start()
            rdma.wait()

            c = (my - 2 - s) % N_DEV
            summed = comm_ref[s + 1] + acc_ref[pl.ds(c * m_per, m_per), :]
            if s < N_DEV - 2:
                comm_ref[s + 1, :, :] = summed
            else:
                out_ref[...] = summed

    return pl.pallas_call(
        body,
        out_shape=jax.ShapeDtypeStruct((m_per, N), jnp.float32),
        in_specs=[
            pl.BlockSpec(memory_space=pltpu.VMEM),
            pl.BlockSpec(memory_space=pltpu.VMEM),
        ],
        out_specs=pl.BlockSpec(memory_space=pltpu.VMEM),
        scratch_shapes=[
            pltpu.VMEM((M, N), jnp.float32),
            pltpu.VMEM((N_DEV, m_per, N), jnp.float32),
            pltpu.SemaphoreType.DMA((N_DEV - 1,)),
            pltpu.SemaphoreType.DMA((N_DEV - 1,)),
        ],
        compiler_params=pltpu.CompilerParams(collective_id=0),
    )(A, B)


# device time: 15730 ns/iter; 2.0188x vs baseline; 2.0188x over previous
import jax
import jax.numpy as jnp
from jax import lax
from jax.experimental import pallas as pl
from jax.experimental.pallas import tpu as pltpu

N_DEV = 4


def kernel(A, B):
    M, K = A.shape
    _, N = B.shape
    m_per = M // N_DEV

    def body(a_ref, b_ref, out_ref, acc_ref, recv_ref, send_sems, recv_sems):
        my = lax.axis_index("i")

        barrier_sem = pltpu.get_barrier_semaphore()
        for off in (1, 2, 3):
            pl.semaphore_signal(
                barrier_sem, inc=1,
                device_id=((my + off) % N_DEV,),
                device_id_type=pl.DeviceIdType.MESH,
            )
        pl.semaphore_wait(barrier_sem, 3)

        a_bf = a_ref[...].astype(jnp.bfloat16)
        b_bf = b_ref[...].astype(jnp.bfloat16)
        part = jnp.dot(a_bf, b_bf, preferred_element_type=jnp.float32)
        acc_ref[...] = part.astype(jnp.bfloat16).reshape(N_DEV, m_per, N)

        rdmas = []
        for off in (1, 2, 3):
            p = (my + off) % N_DEV
            rdma = pltpu.make_async_remote_copy(
                src_ref=acc_ref.at[p],
                dst_ref=recv_ref.at[my],
                send_sem=send_sems.at[off - 1],
                recv_sem=recv_sems.at[off - 1],
                device_id=(p,),
                device_id_type=pl.DeviceIdType.MESH,
            )
            rdma.start()
            rdmas.append(rdma)

        recv_ref[my, :, :] = acc_ref[my]

        for rdma in rdmas:
            rdma.wait_recv()

        out_ref[...] = jnp.sum(recv_ref[...].astype(jnp.float32), axis=0)

        for rdma in rdmas:
            rdma.wait_send()

    return pl.pallas_call(
        body,
        out_shape=jax.ShapeDtypeStruct((m_per, N), jnp.float32),
        in_specs=[
            pl.BlockSpec(memory_space=pltpu.VMEM),
            pl.BlockSpec(memory_space=pltpu.VMEM),
        ],
        out_specs=pl.BlockSpec(memory_space=pltpu.VMEM),
        scratch_shapes=[
            pltpu.VMEM((N_DEV, m_per, N), jnp.bfloat16),
            pltpu.VMEM((N_DEV, m_per, N), jnp.bfloat16),
            pltpu.SemaphoreType.DMA((N_DEV - 1,)),
            pltpu.SemaphoreType.DMA((N_DEV - 1,)),
        ],
        compiler_params=pltpu.CompilerParams(collective_id=0),
    )(A, B)


# device time: 15292 ns/iter; 2.0766x vs baseline; 1.0286x over previous
import jax
import jax.numpy as jnp
from jax import lax
from jax.experimental import pallas as pl
from jax.experimental.pallas import tpu as pltpu

N_DEV = 4


def kernel(A, B):
    M, K = A.shape
    _, N = B.shape
    m_per = M // N_DEV

    def body(a_ref, b_ref, out_ref, acc_ref, recv_ref, send_sems, recv_sems):
        my = lax.axis_index("i")

        barrier_sem = pltpu.get_barrier_semaphore()
        for off in (1, 2, 3):
            pl.semaphore_signal(
                barrier_sem, inc=1,
                device_id=((my + off) % N_DEV,),
                device_id_type=pl.DeviceIdType.MESH,
            )
        pl.semaphore_wait(barrier_sem, 3)

        b_bf = b_ref[...].astype(jnp.bfloat16)

        rdmas = []
        for slot, off in enumerate((2, 1, 3)):
            p = (my + off) % N_DEV
            a_chunk = a_ref[pl.ds(p * m_per, m_per), :].astype(jnp.bfloat16)
            acc_ref[slot, :, :] = jnp.dot(
                a_chunk, b_bf, preferred_element_type=jnp.float32
            ).astype(jnp.bfloat16)
            rdma = pltpu.make_async_remote_copy(
                src_ref=acc_ref.at[slot],
                dst_ref=recv_ref.at[my],
                send_sem=send_sems.at[off - 1],
                recv_sem=recv_sems.at[off - 1],
                device_id=(p,),
                device_id_type=pl.DeviceIdType.MESH,
            )
            rdma.start()
            rdmas.append(rdma)

        a_own = a_ref[pl.ds(my * m_per, m_per), :].astype(jnp.bfloat16)
        recv_ref[my, :, :] = jnp.dot(
            a_own, b_bf, preferred_element_type=jnp.float32
        ).astype(jnp.bfloat16)

        for rdma in rdmas:
            rdma.wait_recv()

        out_ref[...] = jnp.sum(recv_ref[...].astype(jnp.float32), axis=0)

        for rdma in rdmas:
            rdma.wait_send()

    return pl.pallas_call(
        body,
        out_shape=jax.ShapeDtypeStruct((m_per, N), jnp.float32),
        in_specs=[
            pl.BlockSpec(memory_space=pltpu.VMEM),
            pl.BlockSpec(memory_space=pltpu.VMEM),
        ],
        out_specs=pl.BlockSpec(memory_space=pltpu.VMEM),
        scratch_shapes=[
            pltpu.VMEM((N_DEV - 1, m_per, N), jnp.bfloat16),
            pltpu.VMEM((N_DEV, m_per, N), jnp.bfloat16),
            pltpu.SemaphoreType.DMA((N_DEV - 1,)),
            pltpu.SemaphoreType.DMA((N_DEV - 1,)),
        ],
        compiler_params=pltpu.CompilerParams(collective_id=0),
    )(A, B)
